# Initial kernel scaffold; baseline (speedup 1.0000x reference)
import jax
import jax.numpy as jnp
from jax.experimental import pallas as pl


def kernel(boxes, scores, reg):
    raise NotImplementedError("write your pallas kernel here")


# speedup vs baseline: 33.8419x; 33.8419x over previous
import functools

import jax
import jax.numpy as jnp
from jax import lax
from jax.experimental import pallas as pl
from jax.experimental.pallas import tpu as pltpu
from jax.experimental.pallas import tpu_sc as plsc

_N = 20000
_IOU = 0.5
_L = 16
_NT = 16
_BLK = 64
_GPB = _BLK // _L
_NP = 20480
_NBLK = _NP // _BLK
_OWN = _NP // _NT
_OWNB = _NBLK // _NT
_GRP2 = _OWN // _L


def _iota16():
    return lax.iota(jnp.int32, 16)


def _splat_i32(x):
    return jnp.zeros((16,), jnp.int32) + x


def _sup_mask(px1, py1, px2, py2, pa, gx1, gy1, gx2, gy2, ga):
    xx1 = jnp.maximum(px1, gx1)
    yy1 = jnp.maximum(py1, gy1)
    xx2 = jnp.minimum(px2, gx2)
    yy2 = jnp.minimum(py2, gy2)
    w = jnp.maximum(jnp.float32(0.0), xx2 - xx1 + jnp.float32(1.0))
    h = jnp.maximum(jnp.float32(0.0), yy2 - yy1 + jnp.float32(1.0))
    inter = w * h
    ovr = inter / (pa + ga - inter + jnp.float32(1e-10))
    return ovr > jnp.float32(_IOU)


def _nms_body(sx1, sy1, sx2, sy2, sar,
              ox1, oy1, ox2, oy2, osc, or0, or1, or2, or3, rank,
              o0, o1, o2, o3, o4, o5, o6, o7, o8, o9,
              cx1v, cy1v, cx2v, cy2v, carv, flv, keepv,
              sb, rb, rankv, i0, i1, i2, i3, i4, i5, i6, i7, i8,
              b0, b1, b2, b3, b4, b5, b6, b7, b8, b9,
              slot_sh, keep_sh):
    t = lax.axis_index("s")
    ii = _iota16()

    def stage(o, _):
        gbase = (o * _NT + t) * _BLK
        lbase = o * _BLK
        pltpu.sync_copy(sx1.at[pl.ds(gbase, _BLK)], cx1v.at[pl.ds(lbase, _BLK)])
        pltpu.sync_copy(sy1.at[pl.ds(gbase, _BLK)], cy1v.at[pl.ds(lbase, _BLK)])
        pltpu.sync_copy(sx2.at[pl.ds(gbase, _BLK)], cx2v.at[pl.ds(lbase, _BLK)])
        pltpu.sync_copy(sy2.at[pl.ds(gbase, _BLK)], cy2v.at[pl.ds(lbase, _BLK)])
        pltpu.sync_copy(sar.at[pl.ds(gbase, _BLK)], carv.at[pl.ds(lbase, _BLK)])
        for g in range(_GPB):
            flv[pl.ds(lbase + g * _L, _L)] = jnp.zeros((16,), jnp.int32)
        return 0

    lax.fori_loop(0, _OWNB, stage, 0)

    def intra_and_publish(b):
        o = b // _NT
        lb = o * _BLK
        cnt = jnp.int32(0)
        for g in range(_GPB):
            base = lb + g * _L
            gx1 = cx1v[pl.ds(base, _L)]
            gy1 = cy1v[pl.ds(base, _L)]
            gx2 = cx2v[pl.ds(base, _L)]
            gy2 = cy2v[pl.ds(base, _L)]
            ga = carv[pl.ds(base, _L)]
            f = flv[pl.ds(base, _L)]

            def apply_s(s, f):
                ssp = _splat_i32(s)
                px1 = plsc.load_gather(sb, [_splat_i32(0), ssp])
                py1 = plsc.load_gather(sb, [_splat_i32(1), ssp])
                px2 = plsc.load_gather(sb, [_splat_i32(2), ssp])
                py2 = plsc.load_gather(sb, [_splat_i32(3), ssp])
                pa = plsc.load_gather(sb, [_splat_i32(4), ssp])
                sup = _sup_mask(px1, py1, px2, py2, pa, gx1, gy1, gx2, gy2, ga)
                return jnp.where(sup & (f == 0), jnp.int32(1), f)

            f = lax.fori_loop(0, cnt, apply_s, f)

            def w_cond(carry):
                f, _ = carry
                return jnp.max(jnp.where(f == 0, 1, 0).astype(jnp.int32)) > 0

            def w_body(carry):
                f, cnt = carry
                lv = plsc.all_reduce_ffs(f == 0)
                idxv = lv + base
                px1 = plsc.load_gather(cx1v, [idxv])
                py1 = plsc.load_gather(cy1v, [idxv])
                px2 = plsc.load_gather(cx2v, [idxv])
                py2 = plsc.load_gather(cy2v, [idxv])
                pa = plsc.load_gather(carv, [idxv])
                csp = _splat_i32(cnt)
                m0 = ii == 0
                plsc.store_scatter(sb, [_splat_i32(0), csp], px1, mask=m0)
                plsc.store_scatter(sb, [_splat_i32(1), csp], py1, mask=m0)
                plsc.store_scatter(sb, [_splat_i32(2), csp], px2, mask=m0)
                plsc.store_scatter(sb, [_splat_i32(3), csp], py2, mask=m0)
                plsc.store_scatter(sb, [_splat_i32(4), csp], pa, mask=m0)
                f = jnp.where(ii == lv, jnp.int32(2), f)
                sup = _sup_mask(px1, py1, px2, py2, pa, gx1, gy1, gx2, gy2, ga)
                f = jnp.where(sup & (ii > lv) & (f == 0), jnp.int32(1), f)
                return (f, cnt + 1)

            f, cnt = lax.while_loop(w_cond, w_body, (f, cnt))
            flv[pl.ds(base, _L)] = f

        sb[5, pl.ds(0, _L)] = jnp.zeros((16,), jnp.float32) + cnt.astype(jnp.float32)
        sl = lax.rem(b, 2)
        pltpu.sync_copy(sb, slot_sh.at[sl])

    def cross(b):
        scnt = rb[5, pl.ds(0, _L)]
        S = jnp.max(scnt).astype(jnp.int32)
        o_start = (b - t + _NT) // _NT

        def blk_body(o, _):
            for g in range(_GPB):
                base = o * _BLK + g * _L
                f0 = flv[pl.ds(base, _L)]
                has_alive = jnp.max(jnp.where(f0 == 0, 1, 0).astype(jnp.int32)) > 0

                @pl.when(has_alive & (S > 0))
                def _():
                    gx1 = cx1v[pl.ds(base, _L)]
                    gy1 = cy1v[pl.ds(base, _L)]
                    gx2 = cx2v[pl.ds(base, _L)]
                    gy2 = cy2v[pl.ds(base, _L)]
                    ga = carv[pl.ds(base, _L)]

                    def sbody(s, f):
                        ssp = _splat_i32(s)
                        px1 = plsc.load_gather(rb, [_splat_i32(0), ssp])
                        py1 = plsc.load_gather(rb, [_splat_i32(1), ssp])
                        px2 = plsc.load_gather(rb, [_splat_i32(2), ssp])
                        py2 = plsc.load_gather(rb, [_splat_i32(3), ssp])
                        pa = plsc.load_gather(rb, [_splat_i32(4), ssp])
                        sup = _sup_mask(px1, py1, px2, py2, pa,
                                        gx1, gy1, gx2, gy2, ga)
                        return jnp.where(sup & (f == 0), jnp.int32(1), f)

                    flv[pl.ds(base, _L)] = lax.fori_loop(0, S, sbody, f0)
            return 0

        lax.fori_loop(o_start, _OWNB, blk_body, 0)

    @pl.when(t == 0)
    def _():
        intra_and_publish(jnp.int32(0))

    def round_body(b, _):
        plsc.subcore_barrier()
        sl = lax.rem(b, 2)
        pltpu.sync_copy(slot_sh.at[sl], rb)
        cross(b)
        nb = b + 1

        @pl.when((nb < _NBLK) & (t == lax.rem(nb, _NT)))
        def _():
            intra_and_publish(nb)

        return 0

    lax.fori_loop(0, _NBLK, round_body, 0)

    def pub_keep(o, _):
        lbase = o * _BLK
        for g in range(_GPB):
            f = flv[pl.ds(lbase + g * _L, _L)]
            keepv[pl.ds(lbase + g * _L, _L)] = jnp.where(
                f == 2, jnp.float32(1.0), jnp.float32(0.0))
        gbase = (o * _NT + t) * _BLK
        pltpu.sync_copy(keepv.at[pl.ds(lbase, _BLK)],
                        keep_sh.at[pl.ds(gbase, _BLK)])
        return 0

    lax.fori_loop(0, _OWNB, pub_keep, 0)
    plsc.subcore_barrier()
    pltpu.sync_copy(keep_sh, keepv)

    ob = t * _OWN
    pltpu.sync_copy(rank.at[pl.ds(ob, _OWN)], rankv)
    pltpu.sync_copy(ox1.at[pl.ds(ob, _OWN)], i0)
    pltpu.sync_copy(oy1.at[pl.ds(ob, _OWN)], i1)
    pltpu.sync_copy(ox2.at[pl.ds(ob, _OWN)], i2)
    pltpu.sync_copy(oy2.at[pl.ds(ob, _OWN)], i3)
    pltpu.sync_copy(osc.at[pl.ds(ob, _OWN)], i4)
    pltpu.sync_copy(or0.at[pl.ds(ob, _OWN)], i5)
    pltpu.sync_copy(or1.at[pl.ds(ob, _OWN)], i6)
    pltpu.sync_copy(or2.at[pl.ds(ob, _OWN)], i7)
    pltpu.sync_copy(or3.at[pl.ds(ob, _OWN)], i8)

    def p2(g, _):
        base = g * _L
        rk = rankv[pl.ds(base, _L)]
        kf = plsc.load_gather(keepv, [rk])
        x1 = i0[pl.ds(base, _L)]
        y1 = i1[pl.ds(base, _L)]
        x2 = i2[pl.ds(base, _L)]
        y2 = i3[pl.ds(base, _L)]
        sc = i4[pl.ds(base, _L)]
        r0 = i5[pl.ds(base, _L)]
        r1 = i6[pl.ds(base, _L)]
        r2 = i7[pl.ds(base, _L)]
        r3 = i8[pl.ds(base, _L)]
        one = jnp.float32(1.0)
        bbw = x2 - x1 + one
        bbh = y2 - y1 + one
        b0[pl.ds(base, _L)] = x1 * kf
        b1[pl.ds(base, _L)] = y1 * kf
        b2[pl.ds(base, _L)] = x2 * kf
        b3[pl.ds(base, _L)] = y2 * kf
        b4[pl.ds(base, _L)] = sc * kf
        b5[pl.ds(base, _L)] = (x1 + r0 * bbw) * kf
        b6[pl.ds(base, _L)] = (y1 + r1 * bbh) * kf
        b7[pl.ds(base, _L)] = (x2 + r2 * bbw) * kf
        b8[pl.ds(base, _L)] = (y2 + r3 * bbh) * kf
        b9[pl.ds(base, _L)] = sc * kf
        return 0

    lax.fori_loop(0, _GRP2, p2, 0)

    pltpu.sync_copy(b0, o0.at[pl.ds(ob, _OWN)])
    pltpu.sync_copy(b1, o1.at[pl.ds(ob, _OWN)])
    pltpu.sync_copy(b2, o2.at[pl.ds(ob, _OWN)])
    pltpu.sync_copy(b3, o3.at[pl.ds(ob, _OWN)])
    pltpu.sync_copy(b4, o4.at[pl.ds(ob, _OWN)])
    pltpu.sync_copy(b5, o5.at[pl.ds(ob, _OWN)])
    pltpu.sync_copy(b6, o6.at[pl.ds(ob, _OWN)])
    pltpu.sync_copy(b7, o7.at[pl.ds(ob, _OWN)])
    pltpu.sync_copy(b8, o8.at[pl.ds(ob, _OWN)])
    pltpu.sync_copy(b9, o9.at[pl.ds(ob, _OWN)])


_f32 = jnp.float32
_out_types = tuple(jax.ShapeDtypeStruct((_NP,), _f32) for _ in range(10))
_scratch = [
    pltpu.VMEM((_OWN,), _f32),
    pltpu.VMEM((_OWN,), _f32),
    pltpu.VMEM((_OWN,), _f32),
    pltpu.VMEM((_OWN,), _f32),
    pltpu.VMEM((_OWN,), _f32),
    pltpu.VMEM((_OWN,), jnp.int32),
    pltpu.VMEM((_NP,), _f32),
    pltpu.VMEM((6, _BLK), _f32),
    pltpu.VMEM((6, _BLK), _f32),
    pltpu.VMEM((_OWN,), jnp.int32),
] + [pltpu.VMEM((_OWN,), _f32) for _ in range(9)] \
  + [pltpu.VMEM((_OWN,), _f32) for _ in range(10)] + [
    pltpu.VMEM_SHARED((2, 6, _BLK), _f32),
    pltpu.VMEM_SHARED((_NP,), _f32),
]

_mesh = plsc.VectorSubcoreMesh(
    core_axis_name="c", subcore_axis_name="s", num_cores=1)

_nms_call = pl.kernel(
    _nms_body, out_type=_out_types, mesh=_mesh, scratch_types=_scratch,
    compiler_params=pltpu.CompilerParams(needs_layout_passes=False))


@jax.jit
def kernel(boxes, scores, reg):
    n = boxes.shape[0]
    order = jnp.argsort(-scores)
    bx1 = boxes[order, 0]
    by1 = boxes[order, 1]
    bx2 = boxes[order, 2]
    by2 = boxes[order, 3]
    areas = (bx2 - bx1 + 1.0) * (by2 - by1 + 1.0)

    pad = _NP - n
    fpad = 1e7 + 10.0 * jnp.arange(pad, dtype=_f32)
    sx1 = jnp.concatenate([bx1, fpad])
    sy1 = jnp.concatenate([by1, jnp.zeros((pad,), _f32)])
    sx2 = jnp.concatenate([bx2, fpad + 1.0])
    sy2 = jnp.concatenate([by2, jnp.ones((pad,), _f32)])
    sar = jnp.concatenate([areas, jnp.full((pad,), 4.0, _f32)])

    zpad = jnp.zeros((pad,), _f32)
    ox1 = jnp.concatenate([boxes[:, 0], zpad])
    oy1 = jnp.concatenate([boxes[:, 1], zpad])
    ox2 = jnp.concatenate([boxes[:, 2], zpad])
    oy2 = jnp.concatenate([boxes[:, 3], zpad])
    osc = jnp.concatenate([scores, zpad])
    or0 = jnp.concatenate([reg[:, 0], zpad])
    or1 = jnp.concatenate([reg[:, 1], zpad])
    or2 = jnp.concatenate([reg[:, 2], zpad])
    or3 = jnp.concatenate([reg[:, 3], zpad])

    rank = jnp.zeros((n,), jnp.int32).at[order].set(
        jnp.arange(n, dtype=jnp.int32))
    rank = jnp.concatenate(
        [rank, jnp.full((pad,), _NP - 1, jnp.int32)])

    outs = _nms_call(sx1, sy1, sx2, sy2, sar,
                     ox1, oy1, ox2, oy2, osc, or0, or1, or2, or3, rank)
    cols = [o[:n] for o in outs]
    return jnp.stack(cols, axis=1)


# speedup vs baseline: 37.2068x; 1.0994x over previous
import functools

import jax
import jax.numpy as jnp
from jax import lax
from jax.experimental import pallas as pl
from jax.experimental.pallas import tpu as pltpu
from jax.experimental.pallas import tpu_sc as plsc

_N = 20000
_IOU = 0.5
_L = 16
_NT = 16
_BLK = 64
_GPB = _BLK // _L
_NP = 20480
_NBLK = _NP // _BLK
_OWN = _NP // _NT
_OWNB = _NBLK // _NT
_GRP2 = _OWN // _L


def _iota16():
    return lax.iota(jnp.int32, 16)


def _splat_i32(x):
    return jnp.zeros((16,), jnp.int32) + x


def _sup_mask(px1, py1, px2, py2, pa, gx1, gy1, gx2, gy2, ga):
    xx1 = jnp.maximum(px1, gx1)
    yy1 = jnp.maximum(py1, gy1)
    xx2 = jnp.minimum(px2, gx2)
    yy2 = jnp.minimum(py2, gy2)
    w = jnp.maximum(jnp.float32(0.0), xx2 - xx1 + jnp.float32(1.0))
    h = yy2 - yy1 + jnp.float32(1.0)
    inter = w * h
    denom = pa + ga - inter + jnp.float32(1e-10)
    return inter - jnp.float32(0.5) * denom > denom * jnp.float32(2.0 ** -25)


def _nms_body(sx1, sy1, sx2, sy2, sar,
              ox1, oy1, ox2, oy2, osc, or0, or1, or2, or3, rank,
              o0, o1, o2, o3, o4, o5, o6, o7, o8, o9,
              cx1v, cy1v, cx2v, cy2v, carv, flv, keepv,
              sb, rb, rankv, i0, i1, i2, i3, i4, i5, i6, i7, i8,
              b0, b1, b2, b3, b4, b5, b6, b7, b8, b9,
              slot_sh, keep_sh):
    t = lax.axis_index("s")
    ii = _iota16()

    def stage(o, _):
        gbase = (o * _NT + t) * _BLK
        lbase = o * _BLK
        pltpu.sync_copy(sx1.at[pl.ds(gbase, _BLK)], cx1v.at[pl.ds(lbase, _BLK)])
        pltpu.sync_copy(sy1.at[pl.ds(gbase, _BLK)], cy1v.at[pl.ds(lbase, _BLK)])
        pltpu.sync_copy(sx2.at[pl.ds(gbase, _BLK)], cx2v.at[pl.ds(lbase, _BLK)])
        pltpu.sync_copy(sy2.at[pl.ds(gbase, _BLK)], cy2v.at[pl.ds(lbase, _BLK)])
        pltpu.sync_copy(sar.at[pl.ds(gbase, _BLK)], carv.at[pl.ds(lbase, _BLK)])
        for g in range(_GPB):
            flv[pl.ds(lbase + g * _L, _L)] = jnp.zeros((16,), jnp.int32)
        return 0

    lax.fori_loop(0, _OWNB, stage, 0)

    def intra_and_publish(b):
        o = b // _NT
        lb = o * _BLK
        far = jnp.zeros((16,), jnp.float32) + jnp.float32(3e8)
        one16 = jnp.zeros((16,), jnp.float32) + jnp.float32(1.0)
        for q in range(_GPB):
            for r in range(5):
                sb[r, pl.ds(q * _L, _L)] = far if r < 4 else one16
        cnt = jnp.int32(0)
        for g in range(_GPB):
            base = lb + g * _L
            gx1 = cx1v[pl.ds(base, _L)]
            gy1 = cy1v[pl.ds(base, _L)]
            gx2 = cx2v[pl.ds(base, _L)]
            gy2 = cy2v[pl.ds(base, _L)]
            ga = carv[pl.ds(base, _L)]
            f = flv[pl.ds(base, _L)]

            def apply_s(s, f):
                ssp = _splat_i32(s)
                px1 = plsc.load_gather(sb, [_splat_i32(0), ssp])
                py1 = plsc.load_gather(sb, [_splat_i32(1), ssp])
                px2 = plsc.load_gather(sb, [_splat_i32(2), ssp])
                py2 = plsc.load_gather(sb, [_splat_i32(3), ssp])
                pa = plsc.load_gather(sb, [_splat_i32(4), ssp])
                sup = _sup_mask(px1, py1, px2, py2, pa, gx1, gy1, gx2, gy2, ga)
                return jnp.where(sup & (f == 0), jnp.int32(1), f)

            f = lax.fori_loop(0, cnt, apply_s, f)

            def w_cond(carry):
                f, _ = carry
                return jnp.max(jnp.where(f == 0, 1, 0).astype(jnp.int32)) > 0

            def w_body(carry):
                f, cnt = carry
                lv = plsc.all_reduce_ffs(f == 0)
                idxv = lv + base
                px1 = plsc.load_gather(cx1v, [idxv])
                py1 = plsc.load_gather(cy1v, [idxv])
                px2 = plsc.load_gather(cx2v, [idxv])
                py2 = plsc.load_gather(cy2v, [idxv])
                pa = plsc.load_gather(carv, [idxv])
                csp = _splat_i32(cnt)
                m0 = ii == 0
                plsc.store_scatter(sb, [_splat_i32(0), csp], px1, mask=m0)
                plsc.store_scatter(sb, [_splat_i32(1), csp], py1, mask=m0)
                plsc.store_scatter(sb, [_splat_i32(2), csp], px2, mask=m0)
                plsc.store_scatter(sb, [_splat_i32(3), csp], py2, mask=m0)
                plsc.store_scatter(sb, [_splat_i32(4), csp], pa, mask=m0)
                f = jnp.where(ii == lv, jnp.int32(2), f)
                sup = _sup_mask(px1, py1, px2, py2, pa, gx1, gy1, gx2, gy2, ga)
                f = jnp.where(sup & (ii > lv) & (f == 0), jnp.int32(1), f)
                return (f, cnt + 1)

            f, cnt = lax.while_loop(w_cond, w_body, (f, cnt))
            flv[pl.ds(base, _L)] = f

        sb[5, pl.ds(0, _L)] = jnp.zeros((16,), jnp.float32) + cnt.astype(jnp.float32)
        sl = lax.rem(b, 2)
        pltpu.sync_copy(sb, slot_sh.at[sl])

    def cross(b):
        scnt = rb[5, pl.ds(0, _L)]
        S = jnp.max(scnt).astype(jnp.int32)
        S4 = (S + 3) // 4
        o_start = (b - t + _NT) // _NT

        def blk_body(o, _):
            for g in range(_GPB):
                base = o * _BLK + g * _L
                f0 = flv[pl.ds(base, _L)]
                alive = f0 == 0
                has_alive = jnp.max(jnp.where(alive, 1, 0).astype(jnp.int32)) > 0

                @pl.when(has_alive & (S > 0))
                def _():
                    gx1 = cx1v[pl.ds(base, _L)]
                    gy1 = cy1v[pl.ds(base, _L)]
                    gx2 = cx2v[pl.ds(base, _L)]
                    gy2 = cy2v[pl.ds(base, _L)]
                    ga = carv[pl.ds(base, _L)]

                    def sbody(j, acc):
                        for u in range(4):
                            ssp = _splat_i32(j * 4 + u)
                            px1 = plsc.load_gather(rb, [_splat_i32(0), ssp])
                            py1 = plsc.load_gather(rb, [_splat_i32(1), ssp])
                            px2 = plsc.load_gather(rb, [_splat_i32(2), ssp])
                            py2 = plsc.load_gather(rb, [_splat_i32(3), ssp])
                            pa = plsc.load_gather(rb, [_splat_i32(4), ssp])
                            acc = acc | _sup_mask(px1, py1, px2, py2, pa,
                                                  gx1, gy1, gx2, gy2, ga)
                        return acc

                    acc = lax.fori_loop(
                        0, S4, sbody, jnp.zeros((16,), jnp.bool_))
                    flv[pl.ds(base, _L)] = jnp.where(
                        acc & alive, jnp.int32(1), f0)
            return 0

        lax.fori_loop(o_start, _OWNB, blk_body, 0)

    @pl.when(t == 0)
    def _():
        intra_and_publish(jnp.int32(0))

    def round_body(b, _):
        plsc.subcore_barrier()
        sl = lax.rem(b, 2)
        pltpu.sync_copy(slot_sh.at[sl], rb)
        cross(b)
        nb = b + 1

        @pl.when((nb < _NBLK) & (t == lax.rem(nb, _NT)))
        def _():
            intra_and_publish(nb)

        return 0

    lax.fori_loop(0, _NBLK, round_body, 0)

    def pub_keep(o, _):
        lbase = o * _BLK
        for g in range(_GPB):
            f = flv[pl.ds(lbase + g * _L, _L)]
            keepv[pl.ds(lbase + g * _L, _L)] = jnp.where(
                f == 2, jnp.float32(1.0), jnp.float32(0.0))
        gbase = (o * _NT + t) * _BLK
        pltpu.sync_copy(keepv.at[pl.ds(lbase, _BLK)],
                        keep_sh.at[pl.ds(gbase, _BLK)])
        return 0

    lax.fori_loop(0, _OWNB, pub_keep, 0)
    plsc.subcore_barrier()
    pltpu.sync_copy(keep_sh, keepv)

    ob = t * _OWN
    pltpu.sync_copy(rank.at[pl.ds(ob, _OWN)], rankv)
    pltpu.sync_copy(ox1.at[pl.ds(ob, _OWN)], i0)
    pltpu.sync_copy(oy1.at[pl.ds(ob, _OWN)], i1)
    pltpu.sync_copy(ox2.at[pl.ds(ob, _OWN)], i2)
    pltpu.sync_copy(oy2.at[pl.ds(ob, _OWN)], i3)
    pltpu.sync_copy(osc.at[pl.ds(ob, _OWN)], i4)
    pltpu.sync_copy(or0.at[pl.ds(ob, _OWN)], i5)
    pltpu.sync_copy(or1.at[pl.ds(ob, _OWN)], i6)
    pltpu.sync_copy(or2.at[pl.ds(ob, _OWN)], i7)
    pltpu.sync_copy(or3.at[pl.ds(ob, _OWN)], i8)

    def p2(g, _):
        base = g * _L
        rk = rankv[pl.ds(base, _L)]
        kf = plsc.load_gather(keepv, [rk])
        x1 = i0[pl.ds(base, _L)]
        y1 = i1[pl.ds(base, _L)]
        x2 = i2[pl.ds(base, _L)]
        y2 = i3[pl.ds(base, _L)]
        sc = i4[pl.ds(base, _L)]
        r0 = i5[pl.ds(base, _L)]
        r1 = i6[pl.ds(base, _L)]
        r2 = i7[pl.ds(base, _L)]
        r3 = i8[pl.ds(base, _L)]
        one = jnp.float32(1.0)
        bbw = x2 - x1 + one
        bbh = y2 - y1 + one
        b0[pl.ds(base, _L)] = x1 * kf
        b1[pl.ds(base, _L)] = y1 * kf
        b2[pl.ds(base, _L)] = x2 * kf
        b3[pl.ds(base, _L)] = y2 * kf
        b4[pl.ds(base, _L)] = sc * kf
        b5[pl.ds(base, _L)] = (x1 + r0 * bbw) * kf
        b6[pl.ds(base, _L)] = (y1 + r1 * bbh) * kf
        b7[pl.ds(base, _L)] = (x2 + r2 * bbw) * kf
        b8[pl.ds(base, _L)] = (y2 + r3 * bbh) * kf
        b9[pl.ds(base, _L)] = sc * kf
        return 0

    lax.fori_loop(0, _GRP2, p2, 0)

    pltpu.sync_copy(b0, o0.at[pl.ds(ob, _OWN)])
    pltpu.sync_copy(b1, o1.at[pl.ds(ob, _OWN)])
    pltpu.sync_copy(b2, o2.at[pl.ds(ob, _OWN)])
    pltpu.sync_copy(b3, o3.at[pl.ds(ob, _OWN)])
    pltpu.sync_copy(b4, o4.at[pl.ds(ob, _OWN)])
    pltpu.sync_copy(b5, o5.at[pl.ds(ob, _OWN)])
    pltpu.sync_copy(b6, o6.at[pl.ds(ob, _OWN)])
    pltpu.sync_copy(b7, o7.at[pl.ds(ob, _OWN)])
    pltpu.sync_copy(b8, o8.at[pl.ds(ob, _OWN)])
    pltpu.sync_copy(b9, o9.at[pl.ds(ob, _OWN)])


_f32 = jnp.float32
_out_types = tuple(jax.ShapeDtypeStruct((_NP,), _f32) for _ in range(10))
_scratch = [
    pltpu.VMEM((_OWN,), _f32),
    pltpu.VMEM((_OWN,), _f32),
    pltpu.VMEM((_OWN,), _f32),
    pltpu.VMEM((_OWN,), _f32),
    pltpu.VMEM((_OWN,), _f32),
    pltpu.VMEM((_OWN,), jnp.int32),
    pltpu.VMEM((_NP,), _f32),
    pltpu.VMEM((6, _BLK), _f32),
    pltpu.VMEM((6, _BLK), _f32),
    pltpu.VMEM((_OWN,), jnp.int32),
] + [pltpu.VMEM((_OWN,), _f32) for _ in range(9)] \
  + [pltpu.VMEM((_OWN,), _f32) for _ in range(10)] + [
    pltpu.VMEM_SHARED((2, 6, _BLK), _f32),
    pltpu.VMEM_SHARED((_NP,), _f32),
]

_mesh = plsc.VectorSubcoreMesh(
    core_axis_name="c", subcore_axis_name="s", num_cores=1)

_nms_call = pl.kernel(
    _nms_body, out_type=_out_types, mesh=_mesh, scratch_types=_scratch,
    compiler_params=pltpu.CompilerParams(needs_layout_passes=False))


@jax.jit
def kernel(boxes, scores, reg):
    n = boxes.shape[0]
    order = jnp.argsort(-scores)
    bx1 = boxes[order, 0]
    by1 = boxes[order, 1]
    bx2 = boxes[order, 2]
    by2 = boxes[order, 3]
    areas = (bx2 - bx1 + 1.0) * (by2 - by1 + 1.0)

    pad = _NP - n
    fpad = 1e7 + 10.0 * jnp.arange(pad, dtype=_f32)
    sx1 = jnp.concatenate([bx1, fpad])
    sy1 = jnp.concatenate([by1, jnp.zeros((pad,), _f32)])
    sx2 = jnp.concatenate([bx2, fpad + 1.0])
    sy2 = jnp.concatenate([by2, jnp.ones((pad,), _f32)])
    sar = jnp.concatenate([areas, jnp.full((pad,), 4.0, _f32)])

    zpad = jnp.zeros((pad,), _f32)
    ox1 = jnp.concatenate([boxes[:, 0], zpad])
    oy1 = jnp.concatenate([boxes[:, 1], zpad])
    ox2 = jnp.concatenate([boxes[:, 2], zpad])
    oy2 = jnp.concatenate([boxes[:, 3], zpad])
    osc = jnp.concatenate([scores, zpad])
    or0 = jnp.concatenate([reg[:, 0], zpad])
    or1 = jnp.concatenate([reg[:, 1], zpad])
    or2 = jnp.concatenate([reg[:, 2], zpad])
    or3 = jnp.concatenate([reg[:, 3], zpad])

    rank = jnp.zeros((n,), jnp.int32).at[order].set(
        jnp.arange(n, dtype=jnp.int32))
    rank = jnp.concatenate(
        [rank, jnp.full((pad,), _NP - 1, jnp.int32)])

    outs = _nms_call(sx1, sy1, sx2, sy2, sar,
                     ox1, oy1, ox2, oy2, osc, or0, or1, or2, or3, rank)
    cols = [o[:n] for o in outs]
    return jnp.stack(cols, axis=1)


# speedup vs baseline: 39.2237x; 1.0542x over previous
import functools

import jax
import jax.numpy as jnp
from jax import lax
from jax.experimental import pallas as pl
from jax.experimental.pallas import tpu as pltpu
from jax.experimental.pallas import tpu_sc as plsc

_N = 20000
_IOU = 0.5
_L = 16
_NT = 16
_BLK = 64
_GPB = _BLK // _L
_NP = 20480
_NBLK = _NP // _BLK
_OWN = _NP // _NT
_OWNB = _NBLK // _NT
_GRP2 = _OWN // _L


def _iota16():
    return lax.iota(jnp.int32, 16)


def _splat_i32(x):
    return jnp.zeros((16,), jnp.int32) + x


def _sup_mask(px1, py1, px2, py2, pa, gx1, gy1, gx2, gy2, ga):
    xx1 = jnp.maximum(px1, gx1)
    yy1 = jnp.maximum(py1, gy1)
    xx2 = jnp.minimum(px2, gx2)
    yy2 = jnp.minimum(py2, gy2)
    w = jnp.maximum(jnp.float32(0.0), xx2 - xx1 + jnp.float32(1.0))
    h = yy2 - yy1 + jnp.float32(1.0)
    inter = w * h
    denom = pa + ga - inter + jnp.float32(1e-10)
    return inter - jnp.float32(0.5) * denom > denom * jnp.float32(2.0 ** -25)


def _nms_body(sx1, sy1, sx2, sy2, sar,
              ox1, oy1, ox2, oy2, osc, or0, or1, or2, or3, rank,
              o0, o1, o2, o3, o4, o5, o6, o7, o8, o9,
              cx1v, cy1v, cx2v, cy2v, carv, flv, keepv,
              sb, rb, rankv, i0, i1, i2, i3, i4, i5, i6, i7, i8,
              b0, b1, b2, b3, b4, b5, b6, b7, b8, b9,
              slot_sh, keep_sh):
    t = lax.axis_index("s")
    ii = _iota16()

    def stage(o, _):
        gbase = (o * _NT + t) * _BLK
        lbase = o * _BLK
        pltpu.sync_copy(sx1.at[pl.ds(gbase, _BLK)], cx1v.at[pl.ds(lbase, _BLK)])
        pltpu.sync_copy(sy1.at[pl.ds(gbase, _BLK)], cy1v.at[pl.ds(lbase, _BLK)])
        pltpu.sync_copy(sx2.at[pl.ds(gbase, _BLK)], cx2v.at[pl.ds(lbase, _BLK)])
        pltpu.sync_copy(sy2.at[pl.ds(gbase, _BLK)], cy2v.at[pl.ds(lbase, _BLK)])
        pltpu.sync_copy(sar.at[pl.ds(gbase, _BLK)], carv.at[pl.ds(lbase, _BLK)])
        for g in range(_GPB):
            flv[pl.ds(lbase + g * _L, _L)] = jnp.zeros((16,), jnp.int32)
        return 0

    lax.fori_loop(0, _OWNB, stage, 0)

    def intra_and_publish(b):
        o = b // _NT
        lb = o * _BLK
        far = jnp.zeros((16,), jnp.float32) + jnp.float32(3e8)
        one16 = jnp.zeros((16,), jnp.float32) + jnp.float32(1.0)
        for q in range(_GPB):
            for r in range(5):
                sb[r, pl.ds(q * _L, _L)] = far if r < 4 else one16
        csp = _splat_i32(0)
        m0 = ii == 0
        for g in range(_GPB):
            base = lb + g * _L
            gx1 = cx1v[pl.ds(base, _L)]
            gy1 = cy1v[pl.ds(base, _L)]
            gx2 = cx2v[pl.ds(base, _L)]
            gy2 = cy2v[pl.ds(base, _L)]
            ga = carv[pl.ds(base, _L)]
            f = flv[pl.ds(base, _L)]

            if g > 0:
                cnt4 = (jnp.max(csp) + 3) // 4

                def apply_s(j, acc):
                    for u in range(4):
                        ssp = _splat_i32(j * 4 + u)
                        px1 = plsc.load_gather(sb, [_splat_i32(0), ssp])
                        py1 = plsc.load_gather(sb, [_splat_i32(1), ssp])
                        px2 = plsc.load_gather(sb, [_splat_i32(2), ssp])
                        py2 = plsc.load_gather(sb, [_splat_i32(3), ssp])
                        pa = plsc.load_gather(sb, [_splat_i32(4), ssp])
                        acc = acc | _sup_mask(px1, py1, px2, py2, pa,
                                              gx1, gy1, gx2, gy2, ga)
                    return acc

                acc = lax.fori_loop(0, cnt4, apply_s,
                                    jnp.zeros((16,), jnp.bool_))
                f = jnp.where(acc & (f == 0), jnp.int32(1), f)

            def lane4(l4, carry):
                f, csp = carry
                for u in range(4):
                    lsp = _splat_i32(l4 * 4 + u)
                    gate = f.at[lsp].get(mode="promise_in_bounds") == 0
                    px1 = gx1.at[lsp].get(mode="promise_in_bounds")
                    py1 = gy1.at[lsp].get(mode="promise_in_bounds")
                    px2 = gx2.at[lsp].get(mode="promise_in_bounds")
                    py2 = gy2.at[lsp].get(mode="promise_in_bounds")
                    pa = ga.at[lsp].get(mode="promise_in_bounds")
                    m = m0 & gate
                    plsc.store_scatter(sb, [_splat_i32(0), csp], px1, mask=m)
                    plsc.store_scatter(sb, [_splat_i32(1), csp], py1, mask=m)
                    plsc.store_scatter(sb, [_splat_i32(2), csp], px2, mask=m)
                    plsc.store_scatter(sb, [_splat_i32(3), csp], py2, mask=m)
                    plsc.store_scatter(sb, [_splat_i32(4), csp], pa, mask=m)
                    f = jnp.where(gate & (ii == lsp), jnp.int32(2), f)
                    sup = _sup_mask(px1, py1, px2, py2, pa,
                                    gx1, gy1, gx2, gy2, ga)
                    f = jnp.where(gate & sup & (ii > lsp) & (f == 0),
                                  jnp.int32(1), f)
                    csp = csp + jnp.where(gate, 1, 0).astype(jnp.int32)
                return (f, csp)

            f, csp = lax.fori_loop(0, 4, lane4,
                                   (f, csp), unroll=True)
            flv[pl.ds(base, _L)] = f

        sb[5, pl.ds(0, _L)] = csp.astype(jnp.float32)
        sl = lax.rem(b, 2)
        pltpu.sync_copy(sb, slot_sh.at[sl])

    def cross(b):
        scnt = rb[5, pl.ds(0, _L)]
        S = jnp.max(scnt).astype(jnp.int32)
        S4 = (S + 3) // 4
        o_start = (b - t + _NT) // _NT

        def blk_body(o, _):
            for g in range(_GPB):
                base = o * _BLK + g * _L
                f0 = flv[pl.ds(base, _L)]
                alive = f0 == 0
                has_alive = jnp.max(jnp.where(alive, 1, 0).astype(jnp.int32)) > 0

                @pl.when(has_alive & (S > 0))
                def _():
                    gx1 = cx1v[pl.ds(base, _L)]
                    gy1 = cy1v[pl.ds(base, _L)]
                    gx2 = cx2v[pl.ds(base, _L)]
                    gy2 = cy2v[pl.ds(base, _L)]
                    ga = carv[pl.ds(base, _L)]

                    def sbody(j, acc):
                        for u in range(4):
                            ssp = _splat_i32(j * 4 + u)
                            px1 = plsc.load_gather(rb, [_splat_i32(0), ssp])
                            py1 = plsc.load_gather(rb, [_splat_i32(1), ssp])
                            px2 = plsc.load_gather(rb, [_splat_i32(2), ssp])
                            py2 = plsc.load_gather(rb, [_splat_i32(3), ssp])
                            pa = plsc.load_gather(rb, [_splat_i32(4), ssp])
                            acc = acc | _sup_mask(px1, py1, px2, py2, pa,
                                                  gx1, gy1, gx2, gy2, ga)
                        return acc

                    acc = lax.fori_loop(
                        0, S4, sbody, jnp.zeros((16,), jnp.bool_))
                    flv[pl.ds(base, _L)] = jnp.where(
                        acc & alive, jnp.int32(1), f0)
            return 0

        lax.fori_loop(o_start, _OWNB, blk_body, 0)

    def round_body(b, _):
        @pl.when(b >= 0)
        def _():
            plsc.subcore_barrier()
            sl = lax.rem(b, 2)
            pltpu.sync_copy(slot_sh.at[sl], rb)
            cross(b)

        nb = b + 1

        @pl.when((nb < _NBLK) & (t == lax.rem(nb, _NT)))
        def _():
            intra_and_publish(nb)

        return 0

    lax.fori_loop(-1, _NBLK, round_body, 0)

    def pub_keep(o, _):
        lbase = o * _BLK
        for g in range(_GPB):
            f = flv[pl.ds(lbase + g * _L, _L)]
            keepv[pl.ds(lbase + g * _L, _L)] = jnp.where(
                f == 2, jnp.float32(1.0), jnp.float32(0.0))
        gbase = (o * _NT + t) * _BLK
        pltpu.sync_copy(keepv.at[pl.ds(lbase, _BLK)],
                        keep_sh.at[pl.ds(gbase, _BLK)])
        return 0

    lax.fori_loop(0, _OWNB, pub_keep, 0)
    plsc.subcore_barrier()
    pltpu.sync_copy(keep_sh, keepv)

    ob = t * _OWN
    pltpu.sync_copy(rank.at[pl.ds(ob, _OWN)], rankv)
    pltpu.sync_copy(ox1.at[pl.ds(ob, _OWN)], i0)
    pltpu.sync_copy(oy1.at[pl.ds(ob, _OWN)], i1)
    pltpu.sync_copy(ox2.at[pl.ds(ob, _OWN)], i2)
    pltpu.sync_copy(oy2.at[pl.ds(ob, _OWN)], i3)
    pltpu.sync_copy(osc.at[pl.ds(ob, _OWN)], i4)
    pltpu.sync_copy(or0.at[pl.ds(ob, _OWN)], i5)
    pltpu.sync_copy(or1.at[pl.ds(ob, _OWN)], i6)
    pltpu.sync_copy(or2.at[pl.ds(ob, _OWN)], i7)
    pltpu.sync_copy(or3.at[pl.ds(ob, _OWN)], i8)

    def p2(g, _):
        base = g * _L
        rk = rankv[pl.ds(base, _L)]
        kf = plsc.load_gather(keepv, [rk])
        x1 = i0[pl.ds(base, _L)]
        y1 = i1[pl.ds(base, _L)]
        x2 = i2[pl.ds(base, _L)]
        y2 = i3[pl.ds(base, _L)]
        sc = i4[pl.ds(base, _L)]
        r0 = i5[pl.ds(base, _L)]
        r1 = i6[pl.ds(base, _L)]
        r2 = i7[pl.ds(base, _L)]
        r3 = i8[pl.ds(base, _L)]
        one = jnp.float32(1.0)
        bbw = x2 - x1 + one
        bbh = y2 - y1 + one
        b0[pl.ds(base, _L)] = x1 * kf
        b1[pl.ds(base, _L)] = y1 * kf
        b2[pl.ds(base, _L)] = x2 * kf
        b3[pl.ds(base, _L)] = y2 * kf
        b4[pl.ds(base, _L)] = sc * kf
        b5[pl.ds(base, _L)] = (x1 + r0 * bbw) * kf
        b6[pl.ds(base, _L)] = (y1 + r1 * bbh) * kf
        b7[pl.ds(base, _L)] = (x2 + r2 * bbw) * kf
        b8[pl.ds(base, _L)] = (y2 + r3 * bbh) * kf
        b9[pl.ds(base, _L)] = sc * kf
        return 0

    lax.fori_loop(0, _GRP2, p2, 0)

    pltpu.sync_copy(b0, o0.at[pl.ds(ob, _OWN)])
    pltpu.sync_copy(b1, o1.at[pl.ds(ob, _OWN)])
    pltpu.sync_copy(b2, o2.at[pl.ds(ob, _OWN)])
    pltpu.sync_copy(b3, o3.at[pl.ds(ob, _OWN)])
    pltpu.sync_copy(b4, o4.at[pl.ds(ob, _OWN)])
    pltpu.sync_copy(b5, o5.at[pl.ds(ob, _OWN)])
    pltpu.sync_copy(b6, o6.at[pl.ds(ob, _OWN)])
    pltpu.sync_copy(b7, o7.at[pl.ds(ob, _OWN)])
    pltpu.sync_copy(b8, o8.at[pl.ds(ob, _OWN)])
    pltpu.sync_copy(b9, o9.at[pl.ds(ob, _OWN)])


_f32 = jnp.float32
_out_types = tuple(jax.ShapeDtypeStruct((_NP,), _f32) for _ in range(10))
_scratch = [
    pltpu.VMEM((_OWN,), _f32),
    pltpu.VMEM((_OWN,), _f32),
    pltpu.VMEM((_OWN,), _f32),
    pltpu.VMEM((_OWN,), _f32),
    pltpu.VMEM((_OWN,), _f32),
    pltpu.VMEM((_OWN,), jnp.int32),
    pltpu.VMEM((_NP,), _f32),
    pltpu.VMEM((6, _BLK), _f32),
    pltpu.VMEM((6, _BLK), _f32),
    pltpu.VMEM((_OWN,), jnp.int32),
] + [pltpu.VMEM((_OWN,), _f32) for _ in range(9)] \
  + [pltpu.VMEM((_OWN,), _f32) for _ in range(10)] + [
    pltpu.VMEM_SHARED((2, 6, _BLK), _f32),
    pltpu.VMEM_SHARED((_NP,), _f32),
]

_mesh = plsc.VectorSubcoreMesh(
    core_axis_name="c", subcore_axis_name="s", num_cores=1)

_nms_call = pl.kernel(
    _nms_body, out_type=_out_types, mesh=_mesh, scratch_types=_scratch,
    compiler_params=pltpu.CompilerParams(needs_layout_passes=False))


@jax.jit
def kernel(boxes, scores, reg):
    n = boxes.shape[0]
    order = jnp.argsort(-scores)
    bx1 = boxes[order, 0]
    by1 = boxes[order, 1]
    bx2 = boxes[order, 2]
    by2 = boxes[order, 3]
    areas = (bx2 - bx1 + 1.0) * (by2 - by1 + 1.0)

    pad = _NP - n
    fpad = 1e7 + 10.0 * jnp.arange(pad, dtype=_f32)
    sx1 = jnp.concatenate([bx1, fpad])
    sy1 = jnp.concatenate([by1, jnp.zeros((pad,), _f32)])
    sx2 = jnp.concatenate([bx2, fpad + 1.0])
    sy2 = jnp.concatenate([by2, jnp.ones((pad,), _f32)])
    sar = jnp.concatenate([areas, jnp.full((pad,), 4.0, _f32)])

    zpad = jnp.zeros((pad,), _f32)
    ox1 = jnp.concatenate([boxes[:, 0], zpad])
    oy1 = jnp.concatenate([boxes[:, 1], zpad])
    ox2 = jnp.concatenate([boxes[:, 2], zpad])
    oy2 = jnp.concatenate([boxes[:, 3], zpad])
    osc = jnp.concatenate([scores, zpad])
    or0 = jnp.concatenate([reg[:, 0], zpad])
    or1 = jnp.concatenate([reg[:, 1], zpad])
    or2 = jnp.concatenate([reg[:, 2], zpad])
    or3 = jnp.concatenate([reg[:, 3], zpad])

    rank = jnp.zeros((n,), jnp.int32).at[order].set(
        jnp.arange(n, dtype=jnp.int32))
    rank = jnp.concatenate(
        [rank, jnp.full((pad,), _NP - 1, jnp.int32)])

    outs = _nms_call(sx1, sy1, sx2, sy2, sar,
                     ox1, oy1, ox2, oy2, osc, or0, or1, or2, or3, rank)
    cols = [o[:n] for o in outs]
    return jnp.stack(cols, axis=1)


# speedup vs baseline: 47.2430x; 1.2044x over previous
import functools

import jax
import jax.numpy as jnp
from jax import lax
from jax.experimental import pallas as pl
from jax.experimental.pallas import tpu as pltpu
from jax.experimental.pallas import tpu_sc as plsc

_N = 20000
_IOU = 0.5
_L = 16
_NT = 16
_BLK = 64
_GPB = _BLK // _L
_NP = 20480
_NBLK = _NP // _BLK
_OWN = _NP // _NT
_OWNB = _NBLK // _NT
_GRP2 = _OWN // _L


def _iota16():
    return lax.iota(jnp.int32, 16)


def _splat_i32(x):
    return jnp.zeros((16,), jnp.int32) + x


def _sup_mask(px1, py1, px2, py2, pa, gx1, gy1, gx2, gy2, ga):
    xx1 = jnp.maximum(px1, gx1)
    yy1 = jnp.maximum(py1, gy1)
    xx2 = jnp.minimum(px2, gx2)
    yy2 = jnp.minimum(py2, gy2)
    w = jnp.maximum(jnp.float32(0.0), xx2 - xx1 + jnp.float32(1.0))
    h = yy2 - yy1 + jnp.float32(1.0)
    inter = w * h
    denom = pa + ga - inter + jnp.float32(1e-10)
    return inter - jnp.float32(0.5) * denom > denom * jnp.float32(2.0 ** -25)


def _nms_body(sx1, sy1, sx2, sy2, sar,
              ox1, oy1, ox2, oy2, osc, or0, or1, or2, or3, rank,
              o0, o1, o2, o3, o4, o5, o6, o7, o8, o9,
              cx1v, cy1v, cx2v, cy2v, carv, flv, keepv,
              sb, rb, rankv, i0, i1, i2, i3, i4, i5, i6, i7, i8,
              b0, b1, b2, b3, b4, b5, b6, b7, b8, b9,
              slot_sh, keep_sh):
    t = lax.axis_index("s")
    ii = _iota16()

    def stage(o, _):
        gbase = (o * _NT + t) * _BLK
        lbase = o * _BLK
        pltpu.sync_copy(sx1.at[pl.ds(gbase, _BLK)], cx1v.at[pl.ds(lbase, _BLK)])
        pltpu.sync_copy(sy1.at[pl.ds(gbase, _BLK)], cy1v.at[pl.ds(lbase, _BLK)])
        pltpu.sync_copy(sx2.at[pl.ds(gbase, _BLK)], cx2v.at[pl.ds(lbase, _BLK)])
        pltpu.sync_copy(sy2.at[pl.ds(gbase, _BLK)], cy2v.at[pl.ds(lbase, _BLK)])
        pltpu.sync_copy(sar.at[pl.ds(gbase, _BLK)], carv.at[pl.ds(lbase, _BLK)])
        for g in range(_GPB):
            flv[pl.ds(lbase + g * _L, _L)] = jnp.zeros((16,), jnp.int32)
        return 0

    lax.fori_loop(0, _OWNB, stage, 0)

    def intra_and_publish(b):
        o = b // _NT
        lb = o * _BLK
        far = jnp.zeros((16,), jnp.float32) + jnp.float32(3e8)
        one16 = jnp.zeros((16,), jnp.float32) + jnp.float32(1.0)
        for q in range(_GPB):
            for r in range(5):
                sb[r, pl.ds(q * _L, _L)] = far if r < 4 else one16
        csp = _splat_i32(0)
        m0 = ii == 0
        for g in range(_GPB):
            base = lb + g * _L
            gx1 = cx1v[pl.ds(base, _L)]
            gy1 = cy1v[pl.ds(base, _L)]
            gx2 = cx2v[pl.ds(base, _L)]
            gy2 = cy2v[pl.ds(base, _L)]
            ga = carv[pl.ds(base, _L)]
            f = flv[pl.ds(base, _L)]

            if g > 0:
                cnt4 = (csp[0] + 3) // 4

                def apply_s(j, acc):
                    for u in range(4):
                        ssp = _splat_i32(j * 4 + u)
                        px1 = plsc.load_gather(sb, [_splat_i32(0), ssp])
                        py1 = plsc.load_gather(sb, [_splat_i32(1), ssp])
                        px2 = plsc.load_gather(sb, [_splat_i32(2), ssp])
                        py2 = plsc.load_gather(sb, [_splat_i32(3), ssp])
                        pa = plsc.load_gather(sb, [_splat_i32(4), ssp])
                        acc = acc | _sup_mask(px1, py1, px2, py2, pa,
                                              gx1, gy1, gx2, gy2, ga)
                    return acc

                acc = lax.fori_loop(0, cnt4, apply_s,
                                    jnp.zeros((16,), jnp.bool_))
                f = jnp.where(acc & (f == 0), jnp.int32(1), f)

            def lane4(l4, carry):
                f, csp = carry
                for u in range(4):
                    lsp = _splat_i32(l4 * 4 + u)
                    gate = f.at[lsp].get(mode="promise_in_bounds") == 0
                    px1 = gx1.at[lsp].get(mode="promise_in_bounds")
                    py1 = gy1.at[lsp].get(mode="promise_in_bounds")
                    px2 = gx2.at[lsp].get(mode="promise_in_bounds")
                    py2 = gy2.at[lsp].get(mode="promise_in_bounds")
                    pa = ga.at[lsp].get(mode="promise_in_bounds")
                    m = m0 & gate
                    plsc.store_scatter(sb, [_splat_i32(0), csp], px1, mask=m)
                    plsc.store_scatter(sb, [_splat_i32(1), csp], py1, mask=m)
                    plsc.store_scatter(sb, [_splat_i32(2), csp], px2, mask=m)
                    plsc.store_scatter(sb, [_splat_i32(3), csp], py2, mask=m)
                    plsc.store_scatter(sb, [_splat_i32(4), csp], pa, mask=m)
                    f = jnp.where(gate & (ii == lsp), jnp.int32(2), f)
                    sup = _sup_mask(px1, py1, px2, py2, pa,
                                    gx1, gy1, gx2, gy2, ga)
                    f = jnp.where(gate & sup & (ii > lsp) & (f == 0),
                                  jnp.int32(1), f)
                    csp = csp + jnp.where(gate, 1, 0).astype(jnp.int32)
                return (f, csp)

            f, csp = lax.fori_loop(0, 4, lane4,
                                   (f, csp), unroll=True)
            flv[pl.ds(base, _L)] = f

        sb[5, pl.ds(0, _L)] = csp.astype(jnp.float32)
        sl = lax.rem(b, 2)
        pltpu.sync_copy(sb, slot_sh.at[sl])

    def cross(b):
        scnt = rb[5, pl.ds(0, _L)]
        S = scnt[0].astype(jnp.int32)
        S4 = (S + 3) // 4
        o_start = (b - t + _NT) // _NT

        def blk_body(o, _):
            for gg in range(_GPB // 2):
                base0 = o * _BLK + 2 * gg * _L
                base1 = base0 + _L
                f0a = flv[pl.ds(base0, _L)]
                f0b = flv[pl.ds(base1, _L)]
                alive_a = f0a == 0
                alive_b = f0b == 0
                npc = plsc.all_reduce_population_count(alive_a | alive_b)
                has_alive = npc[0] > 0

                @pl.when(has_alive & (S > 0))
                def _():
                    ax1 = cx1v[pl.ds(base0, _L)]
                    ay1 = cy1v[pl.ds(base0, _L)]
                    ax2 = cx2v[pl.ds(base0, _L)]
                    ay2 = cy2v[pl.ds(base0, _L)]
                    aa = carv[pl.ds(base0, _L)]
                    bx1 = cx1v[pl.ds(base1, _L)]
                    by1 = cy1v[pl.ds(base1, _L)]
                    bx2 = cx2v[pl.ds(base1, _L)]
                    by2 = cy2v[pl.ds(base1, _L)]
                    ba = carv[pl.ds(base1, _L)]

                    def sbody(j, carry):
                        acca, accb = carry
                        for u in range(4):
                            ssp = _splat_i32(j * 4 + u)
                            px1 = plsc.load_gather(rb, [_splat_i32(0), ssp])
                            py1 = plsc.load_gather(rb, [_splat_i32(1), ssp])
                            px2 = plsc.load_gather(rb, [_splat_i32(2), ssp])
                            py2 = plsc.load_gather(rb, [_splat_i32(3), ssp])
                            pa = plsc.load_gather(rb, [_splat_i32(4), ssp])
                            acca = acca | _sup_mask(px1, py1, px2, py2, pa,
                                                    ax1, ay1, ax2, ay2, aa)
                            accb = accb | _sup_mask(px1, py1, px2, py2, pa,
                                                    bx1, by1, bx2, by2, ba)
                        return (acca, accb)

                    z = jnp.zeros((16,), jnp.bool_)
                    acca, accb = lax.fori_loop(0, S4, sbody, (z, z))
                    flv[pl.ds(base0, _L)] = jnp.where(
                        acca & alive_a, jnp.int32(1), f0a)
                    flv[pl.ds(base1, _L)] = jnp.where(
                        accb & alive_b, jnp.int32(1), f0b)
            return 0

        lax.fori_loop(o_start, _OWNB, blk_body, 0)

    def round_body(b, _):
        @pl.when(b >= 0)
        def _():
            plsc.subcore_barrier()
            sl = lax.rem(b, 2)
            pltpu.sync_copy(slot_sh.at[sl], rb)
            cross(b)

        nb = b + 1

        @pl.when((nb < _NBLK) & (t == lax.rem(nb, _NT)))
        def _():
            intra_and_publish(nb)

        return 0

    lax.fori_loop(-1, _NBLK, round_body, 0)

    def pub_keep(o, _):
        lbase = o * _BLK
        for g in range(_GPB):
            f = flv[pl.ds(lbase + g * _L, _L)]
            keepv[pl.ds(lbase + g * _L, _L)] = jnp.where(
                f == 2, jnp.float32(1.0), jnp.float32(0.0))
        gbase = (o * _NT + t) * _BLK
        pltpu.sync_copy(keepv.at[pl.ds(lbase, _BLK)],
                        keep_sh.at[pl.ds(gbase, _BLK)])
        return 0

    lax.fori_loop(0, _OWNB, pub_keep, 0)
    plsc.subcore_barrier()
    pltpu.sync_copy(keep_sh, keepv)

    ob = t * _OWN
    pltpu.sync_copy(rank.at[pl.ds(ob, _OWN)], rankv)
    pltpu.sync_copy(ox1.at[pl.ds(ob, _OWN)], i0)
    pltpu.sync_copy(oy1.at[pl.ds(ob, _OWN)], i1)
    pltpu.sync_copy(ox2.at[pl.ds(ob, _OWN)], i2)
    pltpu.sync_copy(oy2.at[pl.ds(ob, _OWN)], i3)
    pltpu.sync_copy(osc.at[pl.ds(ob, _OWN)], i4)
    pltpu.sync_copy(or0.at[pl.ds(ob, _OWN)], i5)
    pltpu.sync_copy(or1.at[pl.ds(ob, _OWN)], i6)
    pltpu.sync_copy(or2.at[pl.ds(ob, _OWN)], i7)
    pltpu.sync_copy(or3.at[pl.ds(ob, _OWN)], i8)

    def p2(g, _):
        base = g * _L
        rk = rankv[pl.ds(base, _L)]
        kf = plsc.load_gather(keepv, [rk])
        x1 = i0[pl.ds(base, _L)]
        y1 = i1[pl.ds(base, _L)]
        x2 = i2[pl.ds(base, _L)]
        y2 = i3[pl.ds(base, _L)]
        sc = i4[pl.ds(base, _L)]
        r0 = i5[pl.ds(base, _L)]
        r1 = i6[pl.ds(base, _L)]
        r2 = i7[pl.ds(base, _L)]
        r3 = i8[pl.ds(base, _L)]
        one = jnp.float32(1.0)
        bbw = x2 - x1 + one
        bbh = y2 - y1 + one
        b0[pl.ds(base, _L)] = x1 * kf
        b1[pl.ds(base, _L)] = y1 * kf
        b2[pl.ds(base, _L)] = x2 * kf
        b3[pl.ds(base, _L)] = y2 * kf
        b4[pl.ds(base, _L)] = sc * kf
        b5[pl.ds(base, _L)] = (x1 + r0 * bbw) * kf
        b6[pl.ds(base, _L)] = (y1 + r1 * bbh) * kf
        b7[pl.ds(base, _L)] = (x2 + r2 * bbw) * kf
        b8[pl.ds(base, _L)] = (y2 + r3 * bbh) * kf
        b9[pl.ds(base, _L)] = sc * kf
        return 0

    lax.fori_loop(0, _GRP2, p2, 0)

    pltpu.sync_copy(b0, o0.at[pl.ds(ob, _OWN)])
    pltpu.sync_copy(b1, o1.at[pl.ds(ob, _OWN)])
    pltpu.sync_copy(b2, o2.at[pl.ds(ob, _OWN)])
    pltpu.sync_copy(b3, o3.at[pl.ds(ob, _OWN)])
    pltpu.sync_copy(b4, o4.at[pl.ds(ob, _OWN)])
    pltpu.sync_copy(b5, o5.at[pl.ds(ob, _OWN)])
    pltpu.sync_copy(b6, o6.at[pl.ds(ob, _OWN)])
    pltpu.sync_copy(b7, o7.at[pl.ds(ob, _OWN)])
    pltpu.sync_copy(b8, o8.at[pl.ds(ob, _OWN)])
    pltpu.sync_copy(b9, o9.at[pl.ds(ob, _OWN)])


_f32 = jnp.float32
_out_types = tuple(jax.ShapeDtypeStruct((_NP,), _f32) for _ in range(10))
_scratch = [
    pltpu.VMEM((_OWN,), _f32),
    pltpu.VMEM((_OWN,), _f32),
    pltpu.VMEM((_OWN,), _f32),
    pltpu.VMEM((_OWN,), _f32),
    pltpu.VMEM((_OWN,), _f32),
    pltpu.VMEM((_OWN,), jnp.int32),
    pltpu.VMEM((_NP,), _f32),
    pltpu.VMEM((6, _BLK), _f32),
    pltpu.VMEM((6, _BLK), _f32),
    pltpu.VMEM((_OWN,), jnp.int32),
] + [pltpu.VMEM((_OWN,), _f32) for _ in range(9)] \
  + [pltpu.VMEM((_OWN,), _f32) for _ in range(10)] + [
    pltpu.VMEM_SHARED((2, 6, _BLK), _f32),
    pltpu.VMEM_SHARED((_NP,), _f32),
]

_mesh = plsc.VectorSubcoreMesh(
    core_axis_name="c", subcore_axis_name="s", num_cores=1)

_nms_call = pl.kernel(
    _nms_body, out_type=_out_types, mesh=_mesh, scratch_types=_scratch,
    compiler_params=pltpu.CompilerParams(needs_layout_passes=False))


@jax.jit
def kernel(boxes, scores, reg):
    n = boxes.shape[0]
    order = jnp.argsort(-scores)
    bx1 = boxes[order, 0]
    by1 = boxes[order, 1]
    bx2 = boxes[order, 2]
    by2 = boxes[order, 3]
    areas = (bx2 - bx1 + 1.0) * (by2 - by1 + 1.0)

    pad = _NP - n
    fpad = 1e7 + 10.0 * jnp.arange(pad, dtype=_f32)
    sx1 = jnp.concatenate([bx1, fpad])
    sy1 = jnp.concatenate([by1, jnp.zeros((pad,), _f32)])
    sx2 = jnp.concatenate([bx2, fpad + 1.0])
    sy2 = jnp.concatenate([by2, jnp.ones((pad,), _f32)])
    sar = jnp.concatenate([areas, jnp.full((pad,), 4.0, _f32)])

    zpad = jnp.zeros((pad,), _f32)
    ox1 = jnp.concatenate([boxes[:, 0], zpad])
    oy1 = jnp.concatenate([boxes[:, 1], zpad])
    ox2 = jnp.concatenate([boxes[:, 2], zpad])
    oy2 = jnp.concatenate([boxes[:, 3], zpad])
    osc = jnp.concatenate([scores, zpad])
    or0 = jnp.concatenate([reg[:, 0], zpad])
    or1 = jnp.concatenate([reg[:, 1], zpad])
    or2 = jnp.concatenate([reg[:, 2], zpad])
    or3 = jnp.concatenate([reg[:, 3], zpad])

    rank = jnp.zeros((n,), jnp.int32).at[order].set(
        jnp.arange(n, dtype=jnp.int32))
    rank = jnp.concatenate(
        [rank, jnp.full((pad,), _NP - 1, jnp.int32)])

    outs = _nms_call(sx1, sy1, sx2, sy2, sar,
                     ox1, oy1, ox2, oy2, osc, or0, or1, or2, or3, rank)
    cols = [o[:n] for o in outs]
    return jnp.stack(cols, axis=1)


# speedup vs baseline: 52.1811x; 1.1045x over previous
import functools

import jax
import jax.numpy as jnp
from jax import lax
from jax.experimental import pallas as pl
from jax.experimental.pallas import tpu as pltpu
from jax.experimental.pallas import tpu_sc as plsc

_N = 20000
_IOU = 0.5
_L = 16
_NT = 16
_BLK = 64
_GPB = _BLK // _L
_NP = 20480
_NBLK = _NP // _BLK
_OWN = _NP // _NT
_OWNB = _NBLK // _NT
_GRP2 = _OWN // _L


def _iota16():
    return lax.iota(jnp.int32, 16)


def _splat_i32(x):
    return jnp.zeros((16,), jnp.int32) + x


def _sup_mask(px1, py1, px2, py2, pa, gx1, gy1, gx2, gy2, ga):
    xx1 = jnp.maximum(px1, gx1)
    yy1 = jnp.maximum(py1, gy1)
    xx2 = jnp.minimum(px2, gx2)
    yy2 = jnp.minimum(py2, gy2)
    w = jnp.maximum(jnp.float32(0.0), xx2 - xx1 + jnp.float32(1.0))
    h = yy2 - yy1 + jnp.float32(1.0)
    inter = w * h
    denom = pa + ga - inter + jnp.float32(1e-10)
    return inter - jnp.float32(0.5) * denom > denom * jnp.float32(2.0 ** -25)


def _nms_body(sx1, sy1, sx2, sy2, sar,
              ox1, oy1, ox2, oy2, osc, or0, or1, or2, or3, rank,
              o0, o1, o2, o3, o4, o5, o6, o7, o8, o9,
              cx1v, cy1v, cx2v, cy2v, carv, flv, keepv,
              sb, rb, rankv, i0, i1, i2, i3, i4, i5, i6, i7, i8,
              b0, b1, b2, b3, b4, b5, b6, b7, b8, b9,
              slot_sh, keep_sh):
    t = lax.axis_index("s")
    ii = _iota16()

    def stage(o, _):
        gbase = (o * _NT + t) * _BLK
        lbase = o * _BLK
        pltpu.sync_copy(sx1.at[pl.ds(gbase, _BLK)], cx1v.at[pl.ds(lbase, _BLK)])
        pltpu.sync_copy(sy1.at[pl.ds(gbase, _BLK)], cy1v.at[pl.ds(lbase, _BLK)])
        pltpu.sync_copy(sx2.at[pl.ds(gbase, _BLK)], cx2v.at[pl.ds(lbase, _BLK)])
        pltpu.sync_copy(sy2.at[pl.ds(gbase, _BLK)], cy2v.at[pl.ds(lbase, _BLK)])
        pltpu.sync_copy(sar.at[pl.ds(gbase, _BLK)], carv.at[pl.ds(lbase, _BLK)])
        for g in range(_GPB):
            flv[pl.ds(lbase + g * _L, _L)] = jnp.zeros((16,), jnp.int32)
        return 0

    lax.fori_loop(0, _OWNB, stage, 0)

    def intra_and_publish(b):
        o = b // _NT
        lb = o * _BLK
        far = jnp.zeros((16,), jnp.float32) + jnp.float32(3e8)
        one16 = jnp.zeros((16,), jnp.float32) + jnp.float32(1.0)
        for q in range(_GPB):
            for r in range(5):
                sb[r, pl.ds(q * _L, _L)] = far if r < 4 else one16
        csp = _splat_i32(0)
        m0 = ii == 0
        for g in range(_GPB):
            base = lb + g * _L
            gx1 = cx1v[pl.ds(base, _L)]
            gy1 = cy1v[pl.ds(base, _L)]
            gx2 = cx2v[pl.ds(base, _L)]
            gy2 = cy2v[pl.ds(base, _L)]
            ga = carv[pl.ds(base, _L)]
            f = flv[pl.ds(base, _L)]

            if g > 0:
                cnt4 = (csp[0] + 3) // 4

                def apply_s(j, acc):
                    for u in range(4):
                        ssp = _splat_i32(j * 4 + u)
                        px1 = plsc.load_gather(sb, [_splat_i32(0), ssp])
                        py1 = plsc.load_gather(sb, [_splat_i32(1), ssp])
                        px2 = plsc.load_gather(sb, [_splat_i32(2), ssp])
                        py2 = plsc.load_gather(sb, [_splat_i32(3), ssp])
                        pa = plsc.load_gather(sb, [_splat_i32(4), ssp])
                        acc = acc | _sup_mask(px1, py1, px2, py2, pa,
                                              gx1, gy1, gx2, gy2, ga)
                    return acc

                acc = lax.fori_loop(0, cnt4, apply_s,
                                    jnp.zeros((16,), jnp.bool_))
                f = jnp.where(acc & (f == 0), jnp.int32(1), f)

            def lane4(l4, carry):
                f, csp = carry
                for u in range(4):
                    lsp = _splat_i32(l4 * 4 + u)
                    gate = f.at[lsp].get(mode="promise_in_bounds") == 0
                    px1 = gx1.at[lsp].get(mode="promise_in_bounds")
                    py1 = gy1.at[lsp].get(mode="promise_in_bounds")
                    px2 = gx2.at[lsp].get(mode="promise_in_bounds")
                    py2 = gy2.at[lsp].get(mode="promise_in_bounds")
                    pa = ga.at[lsp].get(mode="promise_in_bounds")
                    m = m0 & gate
                    plsc.store_scatter(sb, [_splat_i32(0), csp], px1, mask=m)
                    plsc.store_scatter(sb, [_splat_i32(1), csp], py1, mask=m)
                    plsc.store_scatter(sb, [_splat_i32(2), csp], px2, mask=m)
                    plsc.store_scatter(sb, [_splat_i32(3), csp], py2, mask=m)
                    plsc.store_scatter(sb, [_splat_i32(4), csp], pa, mask=m)
                    f = jnp.where(gate & (ii == lsp), jnp.int32(2), f)
                    sup = _sup_mask(px1, py1, px2, py2, pa,
                                    gx1, gy1, gx2, gy2, ga)
                    f = jnp.where(gate & sup & (ii > lsp) & (f == 0),
                                  jnp.int32(1), f)
                    csp = csp + jnp.where(gate, 1, 0).astype(jnp.int32)
                return (f, csp)

            f, csp = lax.fori_loop(0, 4, lane4,
                                   (f, csp), unroll=True)
            flv[pl.ds(base, _L)] = f

        sb[5, pl.ds(0, _L)] = csp.astype(jnp.float32)
        sl = lax.rem(b, 2)
        pltpu.sync_copy(sb, slot_sh.at[sl])

    def cross(b):
        scnt = rb[5, pl.ds(0, _L)]
        S = scnt[0].astype(jnp.int32)
        S2 = (S + 1) // 2
        o_start = (b - t + _NT) // _NT

        def blk_body(o, _):
            bb = o * _BLK
            f0s = [flv[pl.ds(bb + g * _L, _L)] for g in range(_GPB)]
            alives = [f0 == 0 for f0 in f0s]
            any01 = alives[0] | alives[1]
            any23 = alives[2] | alives[3]
            npc = plsc.all_reduce_population_count(any01 | any23)
            has_alive = npc[0] > 0

            @pl.when(has_alive & (S > 0))
            def _():
                cs = [(cx1v[pl.ds(bb + g * _L, _L)],
                       cy1v[pl.ds(bb + g * _L, _L)],
                       cx2v[pl.ds(bb + g * _L, _L)],
                       cy2v[pl.ds(bb + g * _L, _L)],
                       carv[pl.ds(bb + g * _L, _L)]) for g in range(_GPB)]

                def sbody(j, accs):
                    for u in range(2):
                        ssp = _splat_i32(j * 2 + u)
                        px1 = plsc.load_gather(rb, [_splat_i32(0), ssp])
                        py1 = plsc.load_gather(rb, [_splat_i32(1), ssp])
                        px2 = plsc.load_gather(rb, [_splat_i32(2), ssp])
                        py2 = plsc.load_gather(rb, [_splat_i32(3), ssp])
                        pa = plsc.load_gather(rb, [_splat_i32(4), ssp])
                        accs = tuple(
                            acc | _sup_mask(px1, py1, px2, py2, pa, *c)
                            for acc, c in zip(accs, cs))
                    return accs

                z = jnp.zeros((16,), jnp.bool_)
                accs = lax.fori_loop(0, S2, sbody, (z, z, z, z))
                for g in range(_GPB):
                    flv[pl.ds(bb + g * _L, _L)] = jnp.where(
                        accs[g] & alives[g], jnp.int32(1), f0s[g])
            return 0

        lax.fori_loop(o_start, _OWNB, blk_body, 0)

    def round_body(b, _):
        @pl.when(b >= 0)
        def _():
            plsc.subcore_barrier()
            sl = lax.rem(b, 2)
            pltpu.sync_copy(slot_sh.at[sl], rb)
            cross(b)

        nb = b + 1

        @pl.when((nb < _NBLK) & (t == lax.rem(nb, _NT)))
        def _():
            intra_and_publish(nb)

        return 0

    lax.fori_loop(-1, _NBLK, round_body, 0)

    def pub_keep(o, _):
        lbase = o * _BLK
        for g in range(_GPB):
            f = flv[pl.ds(lbase + g * _L, _L)]
            keepv[pl.ds(lbase + g * _L, _L)] = jnp.where(
                f == 2, jnp.float32(1.0), jnp.float32(0.0))
        gbase = (o * _NT + t) * _BLK
        pltpu.sync_copy(keepv.at[pl.ds(lbase, _BLK)],
                        keep_sh.at[pl.ds(gbase, _BLK)])
        return 0

    lax.fori_loop(0, _OWNB, pub_keep, 0)
    plsc.subcore_barrier()
    pltpu.sync_copy(keep_sh, keepv)

    ob = t * _OWN
    pltpu.sync_copy(rank.at[pl.ds(ob, _OWN)], rankv)
    pltpu.sync_copy(ox1.at[pl.ds(ob, _OWN)], i0)
    pltpu.sync_copy(oy1.at[pl.ds(ob, _OWN)], i1)
    pltpu.sync_copy(ox2.at[pl.ds(ob, _OWN)], i2)
    pltpu.sync_copy(oy2.at[pl.ds(ob, _OWN)], i3)
    pltpu.sync_copy(osc.at[pl.ds(ob, _OWN)], i4)
    pltpu.sync_copy(or0.at[pl.ds(ob, _OWN)], i5)
    pltpu.sync_copy(or1.at[pl.ds(ob, _OWN)], i6)
    pltpu.sync_copy(or2.at[pl.ds(ob, _OWN)], i7)
    pltpu.sync_copy(or3.at[pl.ds(ob, _OWN)], i8)

    def p2(g, _):
        base = g * _L
        rk = rankv[pl.ds(base, _L)]
        kf = plsc.load_gather(keepv, [rk])
        x1 = i0[pl.ds(base, _L)]
        y1 = i1[pl.ds(base, _L)]
        x2 = i2[pl.ds(base, _L)]
        y2 = i3[pl.ds(base, _L)]
        sc = i4[pl.ds(base, _L)]
        r0 = i5[pl.ds(base, _L)]
        r1 = i6[pl.ds(base, _L)]
        r2 = i7[pl.ds(base, _L)]
        r3 = i8[pl.ds(base, _L)]
        one = jnp.float32(1.0)
        bbw = x2 - x1 + one
        bbh = y2 - y1 + one
        b0[pl.ds(base, _L)] = x1 * kf
        b1[pl.ds(base, _L)] = y1 * kf
        b2[pl.ds(base, _L)] = x2 * kf
        b3[pl.ds(base, _L)] = y2 * kf
        b4[pl.ds(base, _L)] = sc * kf
        b5[pl.ds(base, _L)] = (x1 + r0 * bbw) * kf
        b6[pl.ds(base, _L)] = (y1 + r1 * bbh) * kf
        b7[pl.ds(base, _L)] = (x2 + r2 * bbw) * kf
        b8[pl.ds(base, _L)] = (y2 + r3 * bbh) * kf
        b9[pl.ds(base, _L)] = sc * kf
        return 0

    lax.fori_loop(0, _GRP2, p2, 0)

    pltpu.sync_copy(b0, o0.at[pl.ds(ob, _OWN)])
    pltpu.sync_copy(b1, o1.at[pl.ds(ob, _OWN)])
    pltpu.sync_copy(b2, o2.at[pl.ds(ob, _OWN)])
    pltpu.sync_copy(b3, o3.at[pl.ds(ob, _OWN)])
    pltpu.sync_copy(b4, o4.at[pl.ds(ob, _OWN)])
    pltpu.sync_copy(b5, o5.at[pl.ds(ob, _OWN)])
    pltpu.sync_copy(b6, o6.at[pl.ds(ob, _OWN)])
    pltpu.sync_copy(b7, o7.at[pl.ds(ob, _OWN)])
    pltpu.sync_copy(b8, o8.at[pl.ds(ob, _OWN)])
    pltpu.sync_copy(b9, o9.at[pl.ds(ob, _OWN)])


_f32 = jnp.float32
_out_types = tuple(jax.ShapeDtypeStruct((_NP,), _f32) for _ in range(10))
_scratch = [
    pltpu.VMEM((_OWN,), _f32),
    pltpu.VMEM((_OWN,), _f32),
    pltpu.VMEM((_OWN,), _f32),
    pltpu.VMEM((_OWN,), _f32),
    pltpu.VMEM((_OWN,), _f32),
    pltpu.VMEM((_OWN,), jnp.int32),
    pltpu.VMEM((_NP,), _f32),
    pltpu.VMEM((6, _BLK), _f32),
    pltpu.VMEM((6, _BLK), _f32),
    pltpu.VMEM((_OWN,), jnp.int32),
] + [pltpu.VMEM((_OWN,), _f32) for _ in range(9)] \
  + [pltpu.VMEM((_OWN,), _f32) for _ in range(10)] + [
    pltpu.VMEM_SHARED((2, 6, _BLK), _f32),
    pltpu.VMEM_SHARED((_NP,), _f32),
]

_mesh = plsc.VectorSubcoreMesh(
    core_axis_name="c", subcore_axis_name="s", num_cores=1)

_nms_call = pl.kernel(
    _nms_body, out_type=_out_types, mesh=_mesh, scratch_types=_scratch,
    compiler_params=pltpu.CompilerParams(needs_layout_passes=False))


@jax.jit
def kernel(boxes, scores, reg):
    n = boxes.shape[0]
    order = jnp.argsort(-scores)
    bx1 = boxes[order, 0]
    by1 = boxes[order, 1]
    bx2 = boxes[order, 2]
    by2 = boxes[order, 3]
    areas = (bx2 - bx1 + 1.0) * (by2 - by1 + 1.0)

    pad = _NP - n
    fpad = 1e7 + 10.0 * jnp.arange(pad, dtype=_f32)
    sx1 = jnp.concatenate([bx1, fpad])
    sy1 = jnp.concatenate([by1, jnp.zeros((pad,), _f32)])
    sx2 = jnp.concatenate([bx2, fpad + 1.0])
    sy2 = jnp.concatenate([by2, jnp.ones((pad,), _f32)])
    sar = jnp.concatenate([areas, jnp.full((pad,), 4.0, _f32)])

    zpad = jnp.zeros((pad,), _f32)
    ox1 = jnp.concatenate([boxes[:, 0], zpad])
    oy1 = jnp.concatenate([boxes[:, 1], zpad])
    ox2 = jnp.concatenate([boxes[:, 2], zpad])
    oy2 = jnp.concatenate([boxes[:, 3], zpad])
    osc = jnp.concatenate([scores, zpad])
    or0 = jnp.concatenate([reg[:, 0], zpad])
    or1 = jnp.concatenate([reg[:, 1], zpad])
    or2 = jnp.concatenate([reg[:, 2], zpad])
    or3 = jnp.concatenate([reg[:, 3], zpad])

    rank = jnp.zeros((n,), jnp.int32).at[order].set(
        jnp.arange(n, dtype=jnp.int32))
    rank = jnp.concatenate(
        [rank, jnp.full((pad,), _NP - 1, jnp.int32)])

    outs = _nms_call(sx1, sy1, sx2, sy2, sar,
                     ox1, oy1, ox2, oy2, osc, or0, or1, or2, or3, rank)
    cols = [o[:n] for o in outs]
    return jnp.stack(cols, axis=1)
